# baseline (speedup 1.0000x reference)
import functools

import jax
import jax.numpy as jnp
from jax import lax
from jax.experimental import pallas as pl
from jax.experimental.pallas import tpu as pltpu
from jax.experimental.pallas import tpu_sc as plsc

_CF = 32


def _tc_body(in_ref, fast_ref):
    fast_ref[...] = in_ref[...]


def _fast_copy(frames):
    B, T, H, W = frames.shape
    return pl.pallas_call(
        _tc_body,
        grid=(B, T // _CF),
        in_specs=[pl.BlockSpec((1, _CF, H, W), lambda b, q: (b, q, 0, 0))],
        out_specs=pl.BlockSpec((1, _CF, H, W), lambda b, q: (b, q, 0, 0)),
        out_shape=jax.ShapeDtypeStruct((B, T, H, W), frames.dtype),
        compiler_params=pltpu.CompilerParams(
            dimension_semantics=("parallel", "parallel"),
        ),
    )(frames)


def _slow_gather(frames):
    B, T, H, W = frames.shape
    Ts = T // 4
    HH = H // 2
    n_items = B * Ts * 2
    n_workers = 32
    per_w = n_items // n_workers

    mesh = plsc.VectorSubcoreMesh(core_axis_name="c", subcore_axis_name="s")

    @functools.partial(
        pl.kernel,
        mesh=mesh,
        out_type=jax.ShapeDtypeStruct((B, Ts, H, W), frames.dtype),
        scratch_types=[
            pltpu.VMEM((2, HH, W), frames.dtype),
            pltpu.SemaphoreType.DMA,
            pltpu.SemaphoreType.DMA,
        ],
    )
    def k(in_hbm, out_hbm, buf, sem_in, sem_out):
        wid = lax.axis_index("s") * 2 + lax.axis_index("c")

        def piece(k_):
            i = wid * per_w + k_
            f, h = i // 2, i % 2
            b, p = f // Ts, f % Ts
            t = (21 * p) // 5
            r0 = h * HH
            return b, p, t, r0

        def start_in(k_, slot):
            b, p, t, r0 = piece(k_)
            d = pltpu.make_async_copy(
                in_hbm.at[b, t, pl.ds(r0, HH)], buf.at[slot], sem_in
            )
            d.start()
            return d

        def start_out(k_, slot):
            b, p, t, r0 = piece(k_)
            d = pltpu.make_async_copy(
                buf.at[slot], out_hbm.at[b, p, pl.ds(r0, HH)], sem_out
            )
            d.start()
            return d

        d_in = start_in(0, 0)
        d_out_prev = None
        for k_ in range(per_w):
            d_in.wait()
            if d_out_prev is not None:
                d_out_prev.wait()
            if k_ + 1 < per_w:
                d_in = start_in(k_ + 1, (k_ + 1) % 2)
            d_out_prev = start_out(k_, k_ % 2)
        d_out_prev.wait()

    return k(frames)


def kernel(frames):
    slow = _slow_gather(frames)
    fast = _fast_copy(frames)
    return (slow, fast)


# speedup vs baseline: 1.6091x; 1.6091x over previous
import jax
import jax.numpy as jnp
from jax.experimental import pallas as pl
from jax.experimental.pallas import tpu as pltpu

_CF = 32
_SPC = _CF // 4


def _body(in_ref, slow_ref, fast_ref):
    q = pl.program_id(1)
    fast_ref[...] = in_ref[...]
    for j in range(_SPC):
        p = _SPC * q + j
        o = (21 * p) // 5 - _CF * q
        slow_ref[:, j : j + 1] = in_ref[:, pl.ds(o, 1)]


def kernel(frames):
    B, T, H, W = frames.shape
    Ts = T // 4

    slow, fast = pl.pallas_call(
        _body,
        grid=(B, T // _CF),
        in_specs=[pl.BlockSpec((1, _CF, H, W), lambda b, q: (b, q, 0, 0))],
        out_specs=(
            pl.BlockSpec((1, _SPC, H, W), lambda b, q: (b, q, 0, 0)),
            pl.BlockSpec((1, _CF, H, W), lambda b, q: (b, q, 0, 0)),
        ),
        out_shape=(
            jax.ShapeDtypeStruct((B, Ts, H, W), frames.dtype),
            jax.ShapeDtypeStruct((B, T, H, W), frames.dtype),
        ),
        compiler_params=pltpu.CompilerParams(
            dimension_semantics=("parallel", "parallel"),
        ),
    )(frames)
    return (slow, fast)


# speedup vs baseline: 1.6112x; 1.0013x over previous
import jax
from jax.experimental import pallas as pl
from jax.experimental.pallas import tpu as pltpu

_CF = 32
_SPC = _CF // 4


def _body(in_ref, slow_ref, fast_ref):
    q = pl.program_id(1)
    fast_ref[...] = in_ref[...]
    for j in range(_SPC):
        p = _SPC * q + j
        o = (21 * p) // 5 - _CF * q
        slow_ref[:, j : j + 1] = in_ref[:, pl.ds(o, 1)]


def kernel(frames):
    B, T, H, W = frames.shape
    Ts = T // 4

    slow, fast = pl.pallas_call(
        _body,
        grid=(B, T // _CF),
        in_specs=[pl.BlockSpec((1, _CF, H, W), lambda b, q: (b, q, 0, 0))],
        out_specs=(
            pl.BlockSpec((1, _SPC, H, W), lambda b, q: (b, q, 0, 0)),
            pl.BlockSpec((1, _CF, H, W), lambda b, q: (b, q, 0, 0)),
        ),
        out_shape=(
            jax.ShapeDtypeStruct((B, Ts, H, W), frames.dtype),
            jax.ShapeDtypeStruct((B, T, H, W), frames.dtype),
        ),
        compiler_params=pltpu.CompilerParams(
            dimension_semantics=("parallel", "parallel"),
        ),
    )(frames)
    return (slow, fast)
